# baseline (speedup 1.0000x reference)
import functools

import jax
import jax.numpy as jnp
from jax.experimental import pallas as pl

_N = 4096
_BR = 1024
_GW = 256
_KNN = 16
_MAX_EDGE = 2.0


def _q_block(xi, xT):
    sqi4 = 0.25 * jnp.sum(xi * xi, axis=1, keepdims=True)
    sqj4 = 0.25 * jnp.sum(xT * xT, axis=0, keepdims=True)
    c = jnp.dot(xi * -0.5, xT, preferred_element_type=jnp.float32)
    return sqi4 + (c + sqj4)


def _dtm_kernel(xi_ref, xT_ref, dtm_ref):
    q = _q_block(xi_ref[...], xT_ref[...])
    g = jnp.minimum(q[:, : _N // 2], q[:, _N // 2 :])
    g = jnp.minimum(g[:, : _N // 4], g[:, _N // 4 :])
    g = jnp.minimum(g[:, : _N // 8], g[:, _N // 8 :])
    g = jnp.minimum(g[:, : _GW], g[:, _GW:]).astype(jnp.bfloat16)
    t = jnp.full((_BR, 1), -jnp.inf, dtype=jnp.bfloat16)
    for _ in range(_KNN):
        t = jnp.min(jnp.where(g > t, g, jnp.bfloat16(jnp.inf)),
                    axis=1, keepdims=True)
    t = t.astype(jnp.float32)
    s16 = float(_KNN) * t + jnp.sum(
        jnp.minimum(q - t, 0.0), axis=1, keepdims=True)
    dtm_ref[...] = jnp.sqrt(jnp.maximum(s16, 0.0)) * 0.5


def _edge_kernel(xi_ref, xT_ref, fi_ref, fjT_ref, out_ref):
    q = jnp.maximum(_q_block(xi_ref[...], xT_ref[...]), 2.5e-13)
    s = q * jax.lax.rsqrt(q)
    ai = 0.5 * fi_ref[...]
    bj = 0.5 * fjT_ref[...]
    m = jnp.maximum(ai, bj)
    e = jnp.maximum(m + m, (ai + bj) + s)
    out_ref[...] = jnp.minimum(e, _MAX_EDGE)


@functools.partial(jax.jit)
def kernel(x):
    xT = x.T
    nblk = _N // _BR
    dtm = pl.pallas_call(
        _dtm_kernel,
        grid=(nblk,),
        in_specs=[
            pl.BlockSpec((_BR, 3), lambda i: (i, 0)),
            pl.BlockSpec((3, _N), lambda i: (0, 0)),
        ],
        out_specs=pl.BlockSpec((_BR, 1), lambda i: (i, 0)),
        out_shape=jax.ShapeDtypeStruct((_N, 1), jnp.float32),
    )(x, xT)
    dtmT = dtm.reshape(1, _N)
    edge = pl.pallas_call(
        _edge_kernel,
        grid=(nblk,),
        in_specs=[
            pl.BlockSpec((_BR, 3), lambda i: (i, 0)),
            pl.BlockSpec((3, _N), lambda i: (0, 0)),
            pl.BlockSpec((_BR, 1), lambda i: (i, 0)),
            pl.BlockSpec((1, _N), lambda i: (0, 0)),
        ],
        out_specs=pl.BlockSpec((_BR, _N), lambda i: (i, 0)),
        out_shape=jax.ShapeDtypeStruct((_N, _N), jnp.float32),
    )(x, xT, dtm, dtmT)
    return edge


# speedup vs baseline: 1.0021x; 1.0021x over previous
import functools

import jax
import jax.numpy as jnp
from jax.experimental import pallas as pl

_N = 4096
_BR = 1024
_GW = 256
_KNN = 16
_MAX_EDGE = 2.0


def _q_block(xi, xT):
    sqi4 = 0.25 * jnp.sum(xi * xi, axis=1, keepdims=True)
    sqj4 = 0.25 * jnp.sum(xT * xT, axis=0, keepdims=True)
    c = jnp.dot(xi * -0.5, xT, preferred_element_type=jnp.float32)
    return sqi4 + (c + sqj4)


def _dtm_kernel(xi_ref, xT_ref, dtm_ref):
    q = _q_block(xi_ref[...], xT_ref[...])
    g = jnp.minimum(q[:, : _N // 2], q[:, _N // 2 :])
    g = jnp.minimum(g[:, : _N // 4], g[:, _N // 4 :])
    g = jnp.minimum(g[:, : _N // 8], g[:, _N // 8 :])
    g = jnp.minimum(g[:, : _GW], g[:, _GW:])
    t = jnp.full((_BR, 1), -jnp.inf, dtype=jnp.float32)
    for _ in range(_KNN):
        t = jnp.min(jnp.where(g > t, g, jnp.inf), axis=1, keepdims=True)
    s16 = float(_KNN) * t + jnp.sum(
        jnp.minimum(q - t, 0.0), axis=1, keepdims=True)
    dtm_ref[...] = jnp.sqrt(jnp.maximum(s16, 0.0)) * 0.5


def _edge_kernel(xi_ref, xT_ref, fi_ref, fjT_ref, out_ref):
    q = jnp.maximum(_q_block(xi_ref[...], xT_ref[...]), 2.5e-13)
    s = q * jax.lax.rsqrt(q)
    ai = 0.5 * fi_ref[...]
    bj = 0.5 * fjT_ref[...]
    m = jnp.maximum(ai, bj)
    e = jnp.maximum(m + m, (ai + bj) + s)
    out_ref[...] = jnp.minimum(e, _MAX_EDGE)


@functools.partial(jax.jit)
def kernel(x):
    xT = x.T
    nblk = _N // _BR
    dtm = pl.pallas_call(
        _dtm_kernel,
        grid=(nblk,),
        in_specs=[
            pl.BlockSpec((_BR, 3), lambda i: (i, 0)),
            pl.BlockSpec((3, _N), lambda i: (0, 0)),
        ],
        out_specs=pl.BlockSpec((_BR, 1), lambda i: (i, 0)),
        out_shape=jax.ShapeDtypeStruct((_N, 1), jnp.float32),
    )(x, xT)
    dtmT = dtm.reshape(1, _N)
    edge = pl.pallas_call(
        _edge_kernel,
        grid=(nblk,),
        in_specs=[
            pl.BlockSpec((_BR, 3), lambda i: (i, 0)),
            pl.BlockSpec((3, _N), lambda i: (0, 0)),
            pl.BlockSpec((_BR, 1), lambda i: (i, 0)),
            pl.BlockSpec((1, _N), lambda i: (0, 0)),
        ],
        out_specs=pl.BlockSpec((_BR, _N), lambda i: (i, 0)),
        out_shape=jax.ShapeDtypeStruct((_N, _N), jnp.float32),
    )(x, xT, dtm, dtmT)
    return edge


# speedup vs baseline: 1.0126x; 1.0105x over previous
import functools

import jax
import jax.numpy as jnp
from jax.experimental import pallas as pl

_N = 4096
_BR = 1024
_BE = 512
_GW = 256
_KNN = 16
_MAX_EDGE = 2.0


def _q_block(xi, xT):
    sqi4 = 0.25 * jnp.sum(xi * xi, axis=1, keepdims=True)
    sqj4 = 0.25 * jnp.sum(xT * xT, axis=0, keepdims=True)
    c = jnp.dot(xi * -0.5, xT, preferred_element_type=jnp.float32)
    return sqi4 + (c + sqj4)


def _dtm_kernel(xi_ref, xT_ref, dtm_ref):
    q = _q_block(xi_ref[...], xT_ref[...])
    g = jnp.minimum(q[:, : _N // 2], q[:, _N // 2 :])
    g = jnp.minimum(g[:, : _N // 4], g[:, _N // 4 :])
    g = jnp.minimum(g[:, : _N // 8], g[:, _N // 8 :])
    g = jnp.minimum(g[:, : _GW], g[:, _GW:])
    t = jnp.full((_BR, 1), -jnp.inf, dtype=jnp.float32)
    for _ in range(_KNN):
        t = jnp.min(jnp.where(g > t, g, jnp.inf), axis=1, keepdims=True)
    s16 = float(_KNN) * t + jnp.sum(
        jnp.minimum(q - t, 0.0), axis=1, keepdims=True)
    dtm_ref[...] = jnp.sqrt(jnp.maximum(s16, 0.0)) * 0.5


def _edge_kernel(xi_ref, xT_ref, fi_ref, fjT_ref, out_ref):
    q = jnp.maximum(_q_block(xi_ref[...], xT_ref[...]), 2.5e-13)
    s = q * jax.lax.rsqrt(q)
    ai = 0.5 * fi_ref[...]
    bj = 0.5 * fjT_ref[...]
    m = jnp.maximum(ai, bj)
    e = jnp.maximum(m + m, (ai + bj) + s)
    out_ref[...] = jnp.minimum(e, _MAX_EDGE)


@functools.partial(jax.jit)
def kernel(x):
    xT = x.T
    nblk = _N // _BR
    dtm = pl.pallas_call(
        _dtm_kernel,
        grid=(nblk,),
        in_specs=[
            pl.BlockSpec((_BR, 3), lambda i: (i, 0)),
            pl.BlockSpec((3, _N), lambda i: (0, 0)),
        ],
        out_specs=pl.BlockSpec((_BR, 1), lambda i: (i, 0)),
        out_shape=jax.ShapeDtypeStruct((_N, 1), jnp.float32),
    )(x, xT)
    dtmT = dtm.reshape(1, _N)
    edge = pl.pallas_call(
        _edge_kernel,
        grid=(_N // _BE,),
        in_specs=[
            pl.BlockSpec((_BE, 3), lambda i: (i, 0)),
            pl.BlockSpec((3, _N), lambda i: (0, 0)),
            pl.BlockSpec((_BE, 1), lambda i: (i, 0)),
            pl.BlockSpec((1, _N), lambda i: (0, 0)),
        ],
        out_specs=pl.BlockSpec((_BE, _N), lambda i: (i, 0)),
        out_shape=jax.ShapeDtypeStruct((_N, _N), jnp.float32),
    )(x, xT, dtm, dtmT)
    return edge
